# Initial kernel scaffold; baseline (speedup 1.0000x reference)
import jax
import jax.numpy as jnp
from jax.experimental import pallas as pl


def kernel(x, table):
    raise NotImplementedError("write your pallas kernel here")


# speedup vs baseline: 4.7611x; 4.7611x over previous
import functools

import jax
import jax.numpy as jnp
from jax import lax
from jax.experimental import pallas as pl
from jax.experimental.pallas import tpu as pltpu
from jax.experimental.pallas import tpu_sc as plsc

NUM_ROWS = 16384
NUM_COLS = 200
EMBED_DIM = 64

NC = 2
NS = 16
NW = NC * NS

B = NUM_ROWS * NUM_COLS
B_PER_W = B // NW
CHUNK = 512
N_CHUNKS = B_PER_W // CHUNK


def _make_kernel():
    mesh = plsc.VectorSubcoreMesh(core_axis_name="c", subcore_axis_name="s")

    @functools.partial(
        pl.kernel,
        mesh=mesh,
        out_type=jax.ShapeDtypeStruct((B, EMBED_DIM), jnp.float32),
        scratch_types=[
            pltpu.VMEM((CHUNK,), jnp.int32),
            pltpu.VMEM((CHUNK, EMBED_DIM), jnp.float32),
            pltpu.SemaphoreType.DMA,
        ],
        compiler_params=pltpu.CompilerParams(use_tc_tiling_on_sc=False),
    )
    def gather_kernel(idx_hbm, table_hbm, out_hbm, idx_v, rows_v, sem):
        wid = lax.axis_index("s") * NC + lax.axis_index("c")
        base = wid * B_PER_W

        def step(g, carry):
            off = base + g * CHUNK
            pltpu.sync_copy(idx_hbm.at[pl.ds(off, CHUNK)], idx_v)
            pltpu.async_copy(table_hbm.at[idx_v], rows_v, sem).wait()
            pltpu.sync_copy(rows_v, out_hbm.at[pl.ds(off, CHUNK)])
            return carry

        lax.fori_loop(0, N_CHUNKS, step, 0)

    return gather_kernel


_gather = _make_kernel()


@jax.jit
def kernel(x, table):
    flat_idx = x.reshape((B,)).astype(jnp.int32)
    out = _gather(flat_idx, table)
    return out.reshape((NUM_ROWS, NUM_COLS, EMBED_DIM))


# speedup vs baseline: 5.1531x; 1.0823x over previous
import functools

import jax
import jax.numpy as jnp
from jax import lax
from jax.experimental import pallas as pl
from jax.experimental.pallas import tpu as pltpu
from jax.experimental.pallas import tpu_sc as plsc

NUM_ROWS = 16384
NUM_COLS = 200
EMBED_DIM = 64

NC = 2
NS = 16
NW = NC * NS

B = NUM_ROWS * NUM_COLS
B_PER_W = B // NW
CHUNK = 800
N_CHUNKS = B_PER_W // CHUNK


def _make_kernel():
    mesh = plsc.VectorSubcoreMesh(core_axis_name="c", subcore_axis_name="s")

    @functools.partial(
        pl.kernel,
        mesh=mesh,
        out_type=jax.ShapeDtypeStruct((B, EMBED_DIM), jnp.float32),
        scratch_types=[
            pltpu.VMEM((CHUNK,), jnp.int32),
            pltpu.VMEM((CHUNK,), jnp.int32),
            pltpu.VMEM((CHUNK, EMBED_DIM), jnp.float32),
            pltpu.VMEM((CHUNK, EMBED_DIM), jnp.float32),
            pltpu.SemaphoreType.DMA,
            pltpu.SemaphoreType.DMA,
            pltpu.SemaphoreType.DMA,
            pltpu.SemaphoreType.DMA,
            pltpu.SemaphoreType.DMA,
            pltpu.SemaphoreType.DMA,
        ],
        compiler_params=pltpu.CompilerParams(use_tc_tiling_on_sc=False),
    )
    def gather_kernel(idx_hbm, table_hbm, out_hbm,
                      idx0, idx1, rows0, rows1,
                      sg0, sg1, ss0, ss1, si0, si1):
        wid = lax.axis_index("s") * NC + lax.axis_index("c")
        base = wid * B_PER_W
        idx_v = (idx0, idx1)
        rows_v = (rows0, rows1)
        sg = (sg0, sg1)
        ss = (ss0, ss1)
        si = (si0, si1)

        pltpu.sync_copy(idx_hbm.at[pl.ds(base, CHUNK)], idx0)
        pltpu.async_copy(table_hbm.at[idx0], rows0, sg0)
        pltpu.async_copy(idx_hbm.at[pl.ds(base + CHUNK, CHUNK)], idx1, si1)

        def pair(i, carry):
            for b in (0, 1):
                g = i * 2 + b
                nb = 1 - b
                off = base + g * CHUNK

                @pl.when(g + 1 < N_CHUNKS)
                def _():
                    pltpu.make_async_copy(
                        idx_hbm.at[pl.ds(off + CHUNK, CHUNK)],
                        idx_v[nb], si[nb]).wait()

                    @pl.when(g >= 1)
                    def _():
                        pltpu.make_async_copy(
                            rows_v[nb],
                            out_hbm.at[pl.ds(off - CHUNK, CHUNK)],
                            ss[nb]).wait()

                    pltpu.async_copy(table_hbm.at[idx_v[nb]], rows_v[nb],
                                     sg[nb])

                pltpu.make_async_copy(table_hbm.at[idx_v[b]], rows_v[b],
                                      sg[b]).wait()

                @pl.when(g + 2 < N_CHUNKS)
                def _():
                    pltpu.async_copy(
                        idx_hbm.at[pl.ds(off + 2 * CHUNK, CHUNK)],
                        idx_v[b], si[b])

                pltpu.async_copy(rows_v[b], out_hbm.at[pl.ds(off, CHUNK)],
                                 ss[b])
            return carry

        lax.fori_loop(0, N_CHUNKS // 2, pair, 0)

        last = base + (N_CHUNKS - 1) * CHUNK
        lb = (N_CHUNKS - 1) % 2
        pltpu.make_async_copy(
            rows_v[1 - lb], out_hbm.at[pl.ds(last - CHUNK, CHUNK)],
            ss[1 - lb]).wait()
        pltpu.make_async_copy(
            rows_v[lb], out_hbm.at[pl.ds(last, CHUNK)], ss[lb]).wait()

    return gather_kernel


_gather = _make_kernel()


@jax.jit
def kernel(x, table):
    flat_idx = x.reshape((B,)).astype(jnp.int32)
    out = _gather(flat_idx, table)
    return out.reshape((NUM_ROWS, NUM_COLS, EMBED_DIM))


# speedup vs baseline: 5.1758x; 1.0044x over previous
import functools

import jax
import jax.numpy as jnp
from jax import lax
from jax.experimental import pallas as pl
from jax.experimental.pallas import tpu as pltpu
from jax.experimental.pallas import tpu_sc as plsc

NUM_ROWS = 16384
NUM_COLS = 200
EMBED_DIM = 64

NC = 2
NS = 16
NW = NC * NS

B = NUM_ROWS * NUM_COLS
B_PER_W = B // NW
CHUNK = 800
ROWS_PER_CHUNK = CHUNK // NUM_COLS
ROWS_PER_W = B_PER_W // NUM_COLS
N_CHUNKS = B_PER_W // CHUNK


def _make_kernel():
    mesh = plsc.VectorSubcoreMesh(core_axis_name="c", subcore_axis_name="s")

    @functools.partial(
        pl.kernel,
        mesh=mesh,
        out_type=jax.ShapeDtypeStruct((NUM_ROWS, NUM_COLS, EMBED_DIM),
                                      jnp.float32),
        scratch_types=[
            pltpu.VMEM((CHUNK,), jnp.int32),
            pltpu.VMEM((CHUNK,), jnp.int32),
            pltpu.VMEM((CHUNK, EMBED_DIM), jnp.float32),
            pltpu.VMEM((CHUNK, EMBED_DIM), jnp.float32),
            pltpu.SemaphoreType.DMA,
            pltpu.SemaphoreType.DMA,
            pltpu.SemaphoreType.DMA,
            pltpu.SemaphoreType.DMA,
            pltpu.SemaphoreType.DMA,
            pltpu.SemaphoreType.DMA,
        ],
        compiler_params=pltpu.CompilerParams(use_tc_tiling_on_sc=False),
    )
    def gather_kernel(idx_hbm, table_hbm, out_hbm,
                      idx0, idx1, rows0, rows1,
                      sg0, sg1, ss0, ss1, si0, si1):
        wid = lax.axis_index("s") * NC + lax.axis_index("c")
        base = wid * B_PER_W
        row_base = wid * ROWS_PER_W
        idx_v = (idx0, idx1)
        rows_v = (rows0, rows1)
        sg = (sg0, sg1)
        ss = (ss0, ss1)
        si = (si0, si1)

        def store_chunk(g, b, sem):
            r0 = row_base + g * ROWS_PER_CHUNK
            for k in range(ROWS_PER_CHUNK):
                pltpu.async_copy(
                    rows_v[b].at[pl.ds(k * NUM_COLS, NUM_COLS), :],
                    out_hbm.at[r0 + k], sem)

        def wait_store_chunk(g, b, sem):
            r0 = row_base + g * ROWS_PER_CHUNK
            for k in range(ROWS_PER_CHUNK):
                pltpu.make_async_copy(
                    rows_v[b].at[pl.ds(k * NUM_COLS, NUM_COLS), :],
                    out_hbm.at[r0 + k], sem).wait()

        pltpu.sync_copy(idx_hbm.at[pl.ds(base, CHUNK)], idx0)
        pltpu.async_copy(table_hbm.at[idx0], rows0, sg0)
        pltpu.async_copy(idx_hbm.at[pl.ds(base + CHUNK, CHUNK)], idx1, si1)

        def pair(i, carry):
            for b in (0, 1):
                g = i * 2 + b
                nb = 1 - b

                @pl.when(g + 1 < N_CHUNKS)
                def _():
                    off = base + g * CHUNK
                    pltpu.make_async_copy(
                        idx_hbm.at[pl.ds(off + CHUNK, CHUNK)],
                        idx_v[nb], si[nb]).wait()

                    @pl.when(g >= 1)
                    def _():
                        wait_store_chunk(g - 1, nb, ss[nb])

                    pltpu.async_copy(table_hbm.at[idx_v[nb]], rows_v[nb],
                                     sg[nb])

                pltpu.make_async_copy(table_hbm.at[idx_v[b]], rows_v[b],
                                      sg[b]).wait()

                @pl.when(g + 2 < N_CHUNKS)
                def _():
                    off = base + g * CHUNK
                    pltpu.async_copy(
                        idx_hbm.at[pl.ds(off + 2 * CHUNK, CHUNK)],
                        idx_v[b], si[b])

                store_chunk(g, b, ss[b])
            return carry

        lax.fori_loop(0, N_CHUNKS // 2, pair, 0)

        lb = (N_CHUNKS - 1) % 2
        wait_store_chunk(N_CHUNKS - 2, 1 - lb, ss[1 - lb])
        wait_store_chunk(N_CHUNKS - 1, lb, ss[lb])

    return gather_kernel


_gather = _make_kernel()


@jax.jit
def kernel(x, table):
    flat_idx = x.reshape((B,)).astype(jnp.int32)
    return _gather(flat_idx, table)
